# Initial kernel scaffold; baseline (speedup 1.0000x reference)
import jax
import jax.numpy as jnp
from jax.experimental import pallas as pl


def kernel(x, edge_index_0, edge_index_1, W_l, W_r, att, bias, Wd_l, Wd_r, att_d, bias_d, gumbel):
    raise NotImplementedError("write your pallas kernel here")


# speedup vs baseline: 12.8460x; 12.8460x over previous
import functools

import jax
import jax.numpy as jnp
from jax import lax
from jax.experimental import pallas as pl
from jax.experimental.pallas import tpu as pltpu
from jax.experimental.pallas import tpu_sc as plsc

N_DEC = 2
TAU = 0.5
NEG_SLOPE = 0.2

NC = 2
NS = 16
B = 128

_AUX_W = 8
_ROWS_BLK = 640


def _lrelu(m):
    return jnp.maximum(m, NEG_SLOPE * m)


def _k1_body(x_ref, wl_ref, wr_ref, att_ref, wdl_ref, wdr_ref, attd_ref,
             xl_ref, xr_ref, c8_ref, xdl8_ref, xdr8_ref):
    xb = x_ref[...]
    xl = jnp.dot(xb, wl_ref[...], preferred_element_type=jnp.float32)
    xr = jnp.dot(xb, wr_ref[...], preferred_element_type=jnp.float32)
    xl_ref[...] = xl
    xr_ref[...] = xr
    c = jnp.dot(_lrelu(xl + xr), att_ref[...],
                preferred_element_type=jnp.float32)
    c8_ref[...] = jnp.broadcast_to(c, (c.shape[0], _AUX_W))
    xdl = jnp.dot(xb, wdl_ref[...], preferred_element_type=jnp.float32)
    xdr = jnp.dot(xb, wdr_ref[...], preferred_element_type=jnp.float32)
    cd = jnp.dot(_lrelu(xdl + xdr), attd_ref[...],
                 preferred_element_type=jnp.float32)
    xdl8_ref[...] = xdl
    xdr8_ref[...] = jnp.concatenate(
        [xdr[:, :2], cd, jnp.zeros((xdr.shape[0], _AUX_W - 3), jnp.float32)],
        axis=1)


def _project(x, W_l, W_r, att, Wd_l, Wd_r, att_d):
    n, d = x.shape
    blk = 1000
    grid = n // blk
    wdl8 = jnp.pad(Wd_l, ((0, 0), (0, _AUX_W - N_DEC)))
    wdr8 = jnp.pad(Wd_r, ((0, 0), (0, _AUX_W - N_DEC)))
    attd8 = jnp.pad(att_d, (0, _AUX_W - N_DEC)).reshape(_AUX_W, 1)
    att1 = att.reshape(d, 1)
    full = lambda shape: pl.BlockSpec(shape, lambda i: (0, 0))
    rows = lambda w: pl.BlockSpec((blk, w), lambda i: (i, 0))
    return pl.pallas_call(
        _k1_body,
        grid=(grid,),
        in_specs=[rows(d), full((d, d)), full((d, d)), full((d, 1)),
                  full((d, _AUX_W)), full((d, _AUX_W)), full((_AUX_W, 1))],
        out_specs=[rows(d), rows(d), rows(_AUX_W), rows(_AUX_W),
                   rows(_AUX_W)],
        out_shape=[jax.ShapeDtypeStruct((n, d), jnp.float32),
                   jax.ShapeDtypeStruct((n, d), jnp.float32),
                   jax.ShapeDtypeStruct((n, _AUX_W), jnp.float32),
                   jax.ShapeDtypeStruct((n, _AUX_W), jnp.float32),
                   jax.ShapeDtypeStruct((n, _AUX_W), jnp.float32)],
    )(x, W_l, W_r, att1, wdl8, wdr8, attd8)


def _make_edge_kernel(n, e, d, with_dec):
    n_chunks = e // B
    mesh = plsc.VectorSubcoreMesh(core_axis_name="c", subcore_axis_name="s",
                                  num_cores=NC, num_subcores=NS)
    out_type = [jax.ShapeDtypeStruct((NC, n, d), jnp.float32),
                jax.ShapeDtypeStruct((NC, n, _AUX_W), jnp.float32)]
    scratch = [
        pltpu.VMEM((B,), jnp.int32),
        pltpu.VMEM((B,), jnp.int32),
        pltpu.VMEM((B, d), jnp.float32),
        pltpu.VMEM((B, d), jnp.float32),
        pltpu.VMEM((B, _AUX_W), jnp.float32),
        pltpu.VMEM((B, _AUX_W), jnp.float32),
        pltpu.VMEM((d,), jnp.float32),
        pltpu.VMEM((16,), jnp.float32),
        pltpu.VMEM((16, 16), jnp.float32),
        pltpu.VMEM((B, _AUX_W), jnp.float32),
        pltpu.VMEM((B, _AUX_W), jnp.float32),
        pltpu.VMEM_SHARED((n, d), jnp.float32),
        pltpu.VMEM_SHARED((n, _AUX_W), jnp.float32),
    ]

    def body(xl_hbm, xr_hbm, c8_hbm, src_hbm, dst_hbm, att_hbm, attd_hbm,
             xdl_hbm, xdr_hbm, z128_hbm, z8_hbm,
             num_out, aux_out,
             src_idx, dst_idx, xl_rows, xr_rows, c8_rows, aux_buf,
             att_v, attd_v, tbuf, xdl_rows, xdr_rows,
             num_sp, aux_sp):
        cid = lax.axis_index("c")
        sid = lax.axis_index("s")
        wid = sid * NC + cid

        @pl.when(sid <= NS - 2)
        def _():
            off = sid * _ROWS_BLK
            pltpu.sync_copy(z128_hbm.at[pl.ds(off, _ROWS_BLK)],
                            num_sp.at[pl.ds(off, _ROWS_BLK)])
            pltpu.sync_copy(z8_hbm.at[pl.ds(off, _ROWS_BLK)],
                            aux_sp.at[pl.ds(off, _ROWS_BLK)])

        tail = n - (NS - 1) * _ROWS_BLK

        @pl.when(sid == NS - 1)
        def _():
            off = (NS - 1) * _ROWS_BLK
            pltpu.sync_copy(z128_hbm.at[pl.ds(off, tail)],
                            num_sp.at[pl.ds(off, tail)])
            pltpu.sync_copy(z8_hbm.at[pl.ds(off, tail)],
                            aux_sp.at[pl.ds(off, tail)])

        pltpu.sync_copy(att_hbm, att_v)
        pltpu.sync_copy(attd_hbm, attd_v)
        plsc.subcore_barrier()

        iota = lax.iota(jnp.int32, 16)
        zero16 = jnp.zeros((16,), jnp.int32)

        for g in range(B // 16):
            eidx = iota + g * 16
            for col in range(4, _AUX_W):
                plsc.store_scatter(aux_buf, [eidx, zero16 + col],
                                   jnp.zeros((16,), jnp.float32))
            if not with_dec:
                for col in range(3):
                    plsc.store_scatter(aux_buf, [eidx, zero16 + col],
                                       jnp.zeros((16,), jnp.float32))

        lo = (wid * n_chunks) // (NC * NS)
        hi = ((wid + 1) * n_chunks) // (NC * NS)

        attv = [att_v[pl.ds(16 * jj, 16)] for jj in range(d // 16)]
        attd_vec = attd_v[...]
        attd0 = attd_vec[0]
        attd1 = attd_vec[1]

        def chunk(j, carry):
            base = j * B
            pltpu.sync_copy(src_hbm.at[pl.ds(base, B)], src_idx)
            pltpu.sync_copy(dst_hbm.at[pl.ds(base, B)], dst_idx)
            pltpu.sync_copy(xl_hbm.at[src_idx], xl_rows)
            pltpu.sync_copy(xr_hbm.at[dst_idx], xr_rows)
            pltpu.sync_copy(c8_hbm.at[dst_idx], c8_rows)
            if with_dec:
                pltpu.sync_copy(xdl_hbm.at[src_idx], xdl_rows)
                pltpu.sync_copy(xdr_hbm.at[dst_idx], xdr_rows)

            def group(g, carry2):
                base16 = g * 16
                eidx = iota + base16
                for jedge in range(16):
                    i = base16 + jedge
                    acc = jnp.zeros((16,), jnp.float32)
                    for jj in range(d // 16):
                        sl = pl.ds(jj * 16, 16)
                        m = xl_rows[i, sl] + xr_rows[i, sl]
                        acc = acc + attv[jj] * _lrelu(m)
                    tbuf[jedge, :] = acc
                e = jnp.zeros((16,), jnp.float32)
                for k in range(16):
                    e = e + plsc.load_gather(tbuf, [iota, zero16 + k])
                cg = plsc.load_gather(c8_rows, [eidx, zero16])
                w = jnp.exp(e - cg)
                plsc.store_scatter(aux_buf, [eidx, zero16 + 3], w)

                if with_dec:
                    xdl0 = plsc.load_gather(xdl_rows, [eidx, zero16])
                    xdl1 = plsc.load_gather(xdl_rows, [eidx, zero16 + 1])
                    xdr0 = plsc.load_gather(xdr_rows, [eidx, zero16])
                    xdr1 = plsc.load_gather(xdr_rows, [eidx, zero16 + 1])
                    cd = plsc.load_gather(xdr_rows, [eidx, zero16 + 2])
                    ed = (attd0 * _lrelu(xdl0 + xdr0) +
                          attd1 * _lrelu(xdl1 + xdr1))
                    wd = jnp.exp(ed - cd)
                    plsc.store_scatter(aux_buf, [eidx, zero16], wd * xdl0)
                    plsc.store_scatter(aux_buf, [eidx, zero16 + 1],
                                       wd * xdl1)
                    plsc.store_scatter(aux_buf, [eidx, zero16 + 2], wd)

                for jedge in range(16):
                    i = base16 + jedge
                    wj = w[jedge]
                    for jj in range(d // 16):
                        sl = pl.ds(jj * 16, 16)
                        xl_rows[i, sl] = wj * xl_rows[i, sl]
                return carry2

            lax.fori_loop(0, B // 16, group, 0)

            pltpu.sync_copy(xl_rows, num_sp.at[dst_idx], add=True)
            pltpu.sync_copy(aux_buf, aux_sp.at[dst_idx], add=True)
            return carry

        lax.fori_loop(lo, hi, chunk, 0)
        plsc.subcore_barrier()

        @pl.when(sid <= NS - 2)
        def _():
            off = sid * _ROWS_BLK
            pltpu.sync_copy(num_sp.at[pl.ds(off, _ROWS_BLK)],
                            num_out.at[cid, pl.ds(off, _ROWS_BLK)])
            pltpu.sync_copy(aux_sp.at[pl.ds(off, _ROWS_BLK)],
                            aux_out.at[cid, pl.ds(off, _ROWS_BLK)])

        @pl.when(sid == NS - 1)
        def _():
            off = (NS - 1) * _ROWS_BLK
            pltpu.sync_copy(num_sp.at[pl.ds(off, tail)],
                            num_out.at[cid, pl.ds(off, tail)])
            pltpu.sync_copy(aux_sp.at[pl.ds(off, tail)],
                            aux_out.at[cid, pl.ds(off, tail)])

    return pl.kernel(body, out_type=out_type, mesh=mesh,
                     scratch_types=scratch,
                     compiler_params=pltpu.CompilerParams(
                         needs_layout_passes=False,
                         use_tc_tiling_on_sc=False))


def _k2_body(xdl8_ref, aux_ref, g8_ref, biasd_ref, dec8_ref):
    aux = aux_ref[0] + aux_ref[1]
    den = 1.0 + aux[:, 2:3]
    logits = ((xdl8_ref[:, :2] + aux[:, :2]) / den + biasd_ref[0, :2]
              + g8_ref[:, :2])
    d0 = (logits[:, 0:1] >= logits[:, 1:2]).astype(jnp.float32)
    dec8_ref[...] = jnp.concatenate(
        [d0, 1.0 - d0, jnp.zeros((d0.shape[0], _AUX_W - 2), jnp.float32)],
        axis=1)


def _decide(xdl8, aux0, gumbel, bias_d):
    n = xdl8.shape[0]
    blk = 1000
    g8 = jnp.pad(gumbel, ((0, 0), (0, _AUX_W - N_DEC)))
    biasd8 = jnp.pad(bias_d, (0, _AUX_W - N_DEC)).reshape(1, _AUX_W)
    return pl.pallas_call(
        _k2_body,
        grid=(n // blk,),
        in_specs=[pl.BlockSpec((blk, _AUX_W), lambda i: (i, 0)),
                  pl.BlockSpec((NC, blk, _AUX_W), lambda i: (0, i, 0)),
                  pl.BlockSpec((blk, _AUX_W), lambda i: (i, 0)),
                  pl.BlockSpec((1, _AUX_W), lambda i: (0, 0))],
        out_specs=pl.BlockSpec((blk, _AUX_W), lambda i: (i, 0)),
        out_shape=jax.ShapeDtypeStruct((n, _AUX_W), jnp.float32),
    )(xdl8, aux0, g8, biasd8)


def _k3_body(xl_ref, n0_ref, a0_ref, n1_ref, a1_ref, dec8_ref, bias_ref,
             out_ref):
    xl = xl_ref[...]
    bias = bias_ref[...]
    dec = dec8_ref[...]
    acc = jnp.zeros_like(xl)
    for s, (nr, ar) in enumerate(((n0_ref, a0_ref), (n1_ref, a1_ref))):
        num = xl + nr[0] + nr[1]
        den = 1.0 + ar[0][:, 3:4] + ar[1][:, 3:4]
        acc = acc + (num / den + bias) * dec[:, s:s + 1]
    out_ref[...] = acc


def _combine(xl, num0, aux0, num1, aux1, dec8, bias):
    n, d = xl.shape
    blk = 1000
    bias2 = bias.reshape(1, d)
    rows = lambda w: pl.BlockSpec((blk, w), lambda i: (i, 0))
    part = lambda w: pl.BlockSpec((NC, blk, w), lambda i: (0, i, 0))
    return pl.pallas_call(
        _k3_body,
        grid=(n // blk,),
        in_specs=[rows(d), part(d), part(_AUX_W), part(d), part(_AUX_W),
                  rows(_AUX_W), pl.BlockSpec((1, d), lambda i: (0, 0))],
        out_specs=rows(d),
        out_shape=jax.ShapeDtypeStruct((n, d), jnp.float32),
    )(xl, num0, aux0, num1, aux1, dec8, bias2)


def kernel(x, edge_index_0, edge_index_1, W_l, W_r, att, bias,
           Wd_l, Wd_r, att_d, bias_d, gumbel):
    n, d = x.shape
    e = edge_index_0.shape[1]

    xl, xr, c8, xdl8, xdr8 = _project(x, W_l, W_r, att, Wd_l, Wd_r, att_d)

    z128 = jnp.zeros((n, d), jnp.float32)
    z8 = jnp.zeros((n, _AUX_W), jnp.float32)
    attd16 = jnp.pad(att_d, (0, 16 - N_DEC))

    edge0 = _make_edge_kernel(n, e, d, with_dec=True)
    edge1 = _make_edge_kernel(n, e, d, with_dec=False)

    num0, aux0 = edge0(xl, xr, c8, edge_index_0[0], edge_index_0[1],
                       att, attd16, xdl8, xdr8, z128, z8)
    num1, aux1 = edge1(xl, xr, c8, edge_index_1[0], edge_index_1[1],
                       att, attd16, xdl8, xdr8, z128, z8)

    dec8 = _decide(xdl8, aux0, gumbel, bias_d)
    out = _combine(xl, num0, aux0, num1, aux1, dec8, bias)
    return out, dec8[:, :N_DEC]
